# baseline (speedup 1.0000x reference)
import jax
import jax.numpy as jnp
from jax.experimental import pallas as pl
from jax.experimental.pallas import tpu as pltpu

_N = 256
_FEATS = 768
_BATCH = 32

_PJ = {1: 32, 2: 64, 4: 128, 8: 8, 16: 16, 32: 1, 64: 2, 128: 4}
_DK = {2: 64, 4: 128, 8: 8, 16: 16, 32: 1, 64: 2, 128: 4}


def _stage(a, k, j):
    n, cols = a.shape
    pj = _PJ[j]
    dk = _DK.get(k)
    if pj >= 8:
        g = n // (2 * pj)
        a4 = a.reshape(g, 2, pj, cols)
        mn = jnp.minimum(a4[:, 0], a4[:, 1]).reshape(n // 2, cols)
        mx = jnp.maximum(a4[:, 0], a4[:, 1]).reshape(n // 2, cols)
        if dk is None:
            nl, nh = mx, mn
        else:
            d = dk // 2 if dk >= 2 * pj else dk
            if d >= 8:
                m7 = mn.reshape(n // (4 * d), 2, d, cols)
                x7 = mx.reshape(n // (4 * d), 2, d, cols)
                nl = jnp.concatenate([x7[:, :1], m7[:, 1:]], axis=1)
                nl = nl.reshape(n // 2, cols)
                nh = jnp.concatenate([m7[:, :1], x7[:, 1:]], axis=1)
                nh = nh.reshape(n // 2, cols)
            else:
                q = jax.lax.broadcasted_iota(jnp.int32, (n // 2, cols), 0)
                ascm = (q & d) != 0
                nl = jnp.where(ascm, mn, mx)
                nh = jnp.where(ascm, mx, mn)
        return jnp.stack(
            [nl.reshape(g, pj, cols), nh.reshape(g, pj, cols)], axis=1
        ).reshape(n, cols)
    row = jax.lax.broadcasted_iota(jnp.int32, (n, cols), 0)
    bitp = (row & pj) != 0
    if pj == 4:
        p = jnp.roll(a.reshape(n // 8, 8, cols), 4, axis=1).reshape(n, cols)
    else:
        p = jnp.where(bitp, jnp.roll(a, pj, axis=0), jnp.roll(a, -pj, axis=0))
    if dk is None:
        take_min = bitp
    else:
        take_min = jnp.logical_xor((row & dk) != 0, bitp)
    return jnp.where(take_min, jnp.minimum(a, p), jnp.maximum(a, p))


def _corr_sort_body(x_ref, o_ref):
    x = x_ref[0]
    a = jax.lax.dot_general(
        x, x, (((0,), (0,)), ((), ())), preferred_element_type=jnp.float32
    )

    k = 2
    while k <= _N:
        j = k // 2
        while j >= 1:
            a = _stage(a, k, j)
            j //= 2
        k *= 2
    n, cols = a.shape
    o_ref[0] = a.reshape(8, 4, 8, cols).transpose(2, 1, 0, 3).reshape(n, cols)


def kernel(x):
    n_bsize, n_feats, n_cols, n_rows = x.shape
    x3 = x.reshape(n_bsize, n_feats, n_cols * n_rows)
    x3 = (x3 * (1.0 / _FEATS) ** 0.5).astype(jnp.bfloat16)
    out = pl.pallas_call(
        _corr_sort_body,
        grid=(n_bsize,),
        in_specs=[pl.BlockSpec((1, n_feats, _N), lambda b: (b, 0, 0))],
        out_specs=pl.BlockSpec((1, _N, _N), lambda b: (b, 0, 0)),
        out_shape=jax.ShapeDtypeStruct((n_bsize, _N, _N), jnp.float32),
    )(x3)
    return out.reshape(n_bsize, _N, n_cols, n_rows)


# speedup vs baseline: 1.1553x; 1.1553x over previous
import jax
import jax.numpy as jnp
from jax.experimental import pallas as pl
from jax.experimental.pallas import tpu as pltpu

_N = 256
_FEATS = 768
_BATCH = 32

_PJ = {1: 32, 2: 64, 4: 128, 8: 8, 16: 16, 32: 1, 64: 2, 128: 4}
_DK = {2: 64, 4: 128, 8: 8, 16: 16, 32: 1, 64: 2, 128: 4}


def _stage(a, k, j):
    n, cols = a.shape
    pj = _PJ[j]
    dk = _DK.get(k)
    if pj >= 8:
        g = n // (2 * pj)
        a4 = a.reshape(g, 2, pj, cols)
        mn = jnp.minimum(a4[:, 0], a4[:, 1]).reshape(n // 2, cols)
        mx = jnp.maximum(a4[:, 0], a4[:, 1]).reshape(n // 2, cols)
        if dk is None:
            nl, nh = mx, mn
        else:
            d = dk // 2 if dk >= 2 * pj else dk
            if d >= 8:
                m7 = mn.reshape(n // (4 * d), 2, d, cols)
                x7 = mx.reshape(n // (4 * d), 2, d, cols)
                nl = jnp.concatenate([x7[:, :1], m7[:, 1:]], axis=1)
                nl = nl.reshape(n // 2, cols)
                nh = jnp.concatenate([m7[:, :1], x7[:, 1:]], axis=1)
                nh = nh.reshape(n // 2, cols)
            else:
                q = jax.lax.broadcasted_iota(jnp.int32, (n // 2, cols), 0)
                ascm = (q & d) != 0
                nl = jnp.where(ascm, mn, mx)
                nh = jnp.where(ascm, mx, mn)
        return jnp.stack(
            [nl.reshape(g, pj, cols), nh.reshape(g, pj, cols)], axis=1
        ).reshape(n, cols)
    row = jax.lax.broadcasted_iota(jnp.int32, (n, cols), 0)
    bitp = (row & pj) != 0
    if pj == 4:
        p = jnp.roll(a.reshape(n // 8, 8, cols), 4, axis=1).reshape(n, cols)
    else:
        p = jnp.where(bitp, jnp.roll(a, pj, axis=0), jnp.roll(a, -pj, axis=0))
    if dk is None:
        take_min = bitp
    else:
        take_min = jnp.logical_xor((row & dk) != 0, bitp)
    return jnp.where(take_min, jnp.minimum(a, p), jnp.maximum(a, p))


_BPS = 2


def _corr_sort_body(x_ref, o_ref):
    for b in range(_BPS):
        x = x_ref[b]
        a = jax.lax.dot_general(
            x, x, (((0,), (0,)), ((), ())), preferred_element_type=jnp.float32
        ) * (1.0 / _FEATS)

        k = 2
        while k <= _N:
            j = k // 2
            while j >= 1:
                a = _stage(a, k, j)
                j //= 2
            k *= 2
        n, cols = a.shape
        o_ref[b] = a.reshape(8, 4, 8, cols).transpose(2, 1, 0, 3).reshape(n, cols)


def kernel(x):
    n_bsize, n_feats, n_cols, n_rows = x.shape
    x3 = x.reshape(n_bsize, n_feats, n_cols * n_rows)
    out = pl.pallas_call(
        _corr_sort_body,
        grid=(n_bsize // _BPS,),
        in_specs=[pl.BlockSpec((_BPS, n_feats, _N), lambda b: (b, 0, 0))],
        out_specs=pl.BlockSpec((_BPS, _N, _N), lambda b: (b, 0, 0)),
        out_shape=jax.ShapeDtypeStruct((n_bsize, _N, _N), jnp.float32),
    )(x3)
    return out.reshape(n_bsize, _N, n_cols, n_rows)
